# baseline (speedup 1.0000x reference)
import jax
import jax.numpy as jnp
from jax.experimental import pallas as pl
from jax.experimental.pallas import tpu as pltpu

_TOKENS = 16384
_N_EMBED = 4096
_N_EXP = 64
_K = 8
_BLK_T = 1024

def _gauss_t(b_route):
    seed = jnp.int32(42) + (jnp.zeros((), jnp.int32) *
                            b_route[0].astype(jnp.int32))
    g = jax.random.normal(
        jax.random.key(seed), (_TOKENS, _N_EXP), dtype=jnp.float32)
    return jnp.transpose(g)


def _router_kernel(x_ref, w_ref, b_ref, g_ref, out_ref, idx_ref):
    acc = jax.lax.dot_general(
        x_ref[...], w_ref[...], (((1,), (0,)), ((), ())),
        precision=jax.lax.Precision.DEFAULT,
        preferred_element_type=jnp.float32)
    acc = acc + b_ref[...]
    acc_t = jnp.transpose(acc)
    logits = acc_t[:_N_EXP, :]
    nlog = acc_t[_N_EXP:, :]
    noisy = logits + g_ref[...] * jax.nn.softplus(nlog)

    eidx = jax.lax.broadcasted_iota(jnp.int32, (_N_EXP, _BLK_T), 0)
    slot = jax.lax.broadcasted_iota(jnp.int32, (_K, _BLK_T), 0)
    work = noisy
    idxs_t = jnp.zeros((_K, _BLK_T), jnp.int32)
    vmax = None
    for j in range(_K):
        m = jnp.max(work, axis=0, keepdims=True)
        if j == 0:
            vmax = m
        sel = work == m
        win = jnp.min(jnp.where(sel, eidx, _N_EXP), axis=0, keepdims=True)
        idxs_t = jnp.where(slot == j, win, idxs_t)
        work = jnp.where(eidx == win, -jnp.inf, work)
    idx_ref[...] = jnp.transpose(idxs_t)

    e = jnp.where(work == -jnp.inf, jnp.exp(noisy - vmax), 0.0)
    sm = e / jnp.sum(e, axis=0, keepdims=True)
    out_ref[...] = jnp.transpose(sm)


def kernel(mh_output, W_route, b_route, W_noise, b_noise):
    w_cat = jnp.concatenate([W_route, W_noise], axis=1)
    b_cat = jnp.concatenate([b_route, b_noise])[None, :]
    grid = (_TOKENS // _BLK_T,)
    router, indices = pl.pallas_call(
        _router_kernel,
        grid=grid,
        in_specs=[
            pl.BlockSpec((_BLK_T, _N_EMBED), lambda t: (t, 0)),
            pl.BlockSpec((_N_EMBED, 2 * _N_EXP), lambda t: (0, 0)),
            pl.BlockSpec((1, 2 * _N_EXP), lambda t: (0, 0)),
            pl.BlockSpec((_N_EXP, _BLK_T), lambda t: (0, t)),
        ],
        out_specs=[
            pl.BlockSpec((_BLK_T, _N_EXP), lambda t: (t, 0)),
            pl.BlockSpec((_BLK_T, _K), lambda t: (t, 0)),
        ],
        out_shape=[
            jax.ShapeDtypeStruct((_TOKENS, _N_EXP), jnp.float32),
            jax.ShapeDtypeStruct((_TOKENS, _K), jnp.int32),
        ],
        compiler_params=pltpu.CompilerParams(
            dimension_semantics=("parallel",)),
    )(mh_output, w_cat, b_cat, _gauss_t(b_route))
    return (router, indices)


# speedup vs baseline: 1.1250x; 1.1250x over previous
import jax
import jax.numpy as jnp
from jax.experimental import pallas as pl
from jax.experimental.pallas import tpu as pltpu

_TOKENS = 16384
_N_EMBED = 4096
_N_EXP = 64
_K = 8
_BLK_T = 1024

def _gauss_t(b_route):
    seed = jnp.int32(42) + (jnp.zeros((), jnp.int32) *
                            b_route[0].astype(jnp.int32))
    g = jax.random.normal(
        jax.random.key(seed), (_TOKENS, _N_EXP), dtype=jnp.float32)
    return jnp.transpose(g)


def _router_kernel(x_ref, w_ref, b_ref, g_ref, out_ref, idx_ref):
    acc = jax.lax.dot_general(
        x_ref[...], w_ref[...], (((1,), (0,)), ((), ())),
        precision=jax.lax.Precision.DEFAULT,
        preferred_element_type=jnp.float32)
    acc = acc + b_ref[...]
    acc_t = jnp.transpose(acc)
    logits = acc_t[:_N_EXP, :]
    nlog = acc_t[_N_EXP:, :]
    noisy = logits + g_ref[...] * jax.nn.softplus(nlog)

    eidx = jax.lax.broadcasted_iota(jnp.int32, (_N_EXP, _BLK_T), 0)
    slot = jax.lax.broadcasted_iota(jnp.int32, (_K, _BLK_T), 0)
    work = noisy
    idxs_t = jnp.zeros((_K, _BLK_T), jnp.int32)
    vmax = None
    for j in range(_K):
        m = jnp.max(work, axis=0, keepdims=True)
        if j == 0:
            vmax = m
        sel = work == m
        win = jnp.min(jnp.where(sel, eidx, _N_EXP), axis=0, keepdims=True)
        idxs_t = jnp.where(slot == j, win, idxs_t)
        work = jnp.where(eidx == win, -jnp.inf, work)
    idx_ref[...] = idxs_t

    e = jnp.where(work == -jnp.inf, jnp.exp(noisy - vmax), 0.0)
    out_ref[...] = e / jnp.sum(e, axis=0, keepdims=True)


def kernel(mh_output, W_route, b_route, W_noise, b_noise):
    w_cat = jnp.concatenate([W_route, W_noise], axis=1)
    b_cat = jnp.concatenate([b_route, b_noise])[None, :]
    grid = (_TOKENS // _BLK_T,)
    router, indices = pl.pallas_call(
        _router_kernel,
        grid=grid,
        in_specs=[
            pl.BlockSpec((_BLK_T, _N_EMBED), lambda t: (t, 0)),
            pl.BlockSpec((_N_EMBED, 2 * _N_EXP), lambda t: (0, 0)),
            pl.BlockSpec((1, 2 * _N_EXP), lambda t: (0, 0)),
            pl.BlockSpec((_N_EXP, _BLK_T), lambda t: (0, t)),
        ],
        out_specs=[
            pl.BlockSpec((_N_EXP, _BLK_T), lambda t: (0, t)),
            pl.BlockSpec((_K, _BLK_T), lambda t: (0, t)),
        ],
        out_shape=[
            jax.ShapeDtypeStruct((_N_EXP, _TOKENS), jnp.float32),
            jax.ShapeDtypeStruct((_K, _TOKENS), jnp.int32),
        ],
        compiler_params=pltpu.CompilerParams(
            dimension_semantics=("parallel",)),
    )(mh_output, w_cat, b_cat, _gauss_t(b_route))
    return (jnp.transpose(router), jnp.transpose(indices))
